# baseline (speedup 1.0000x reference)
import jax
import jax.numpy as jnp
from jax import lax
from jax.experimental import pallas as pl
from jax.experimental.pallas import tpu as pltpu
from jax.experimental.pallas import tpu_sc as plsc

_N = 10000
_E = 320000
_D = 128
_R = 3
_G = 100
_NP = 10240
_NC = 2
_NT = 16
_NH = _N // _NC
_NHP = 5120
_CH = 125
_ESC = 176000
_EPT = _ESC // _NT
_NCHUNK = _EPT // _CH
_NDUMP = 128
_ROWS = _NHP + _NDUMP
_ZPT = _ROWS // _NT
_OPT = _NHP // _NT
_NB = 1000
_NBLK = _N // _NB
_NBP = 1024
_TBLK = _NP // _NBP


def _prep_body(x_ref, wr_ref, tw_ref, tv_ref):
    x = x_ref[...]
    tw_ref[0] = jnp.dot(x, wr_ref[0, 0], preferred_element_type=jnp.float32)
    tv_ref[0] = jnp.dot(x, wr_ref[1, 0], preferred_element_type=jnp.float32)


def _tc_prep(x, wrel2):
    return pl.pallas_call(
        _prep_body,
        grid=(_R, _TBLK),
        in_specs=[
            pl.BlockSpec((_NBP, _D), lambda r, i: (i, 0)),
            pl.BlockSpec((2, 1, _D, _D), lambda r, i: (0, r, 0, 0)),
        ],
        out_specs=[
            pl.BlockSpec((1, _NBP, _D), lambda r, i: (r, i, 0)),
            pl.BlockSpec((1, _NBP, _D), lambda r, i: (r, i, 0)),
        ],
        out_shape=[
            jax.ShapeDtypeStruct((_R, _NP, _D), jnp.float32),
            jax.ShapeDtypeStruct((_R, _NP, _D), jnp.float32),
        ],
    )(x, wrel2)


def _sc_body(tab, gidx, sidx, zrows, out,
             gidx_v, sidx_v, rows_a, rows_b, acc, sem_a, sem_b):
    c = lax.axis_index("c")
    s = lax.axis_index("s")

    pltpu.sync_copy(zrows, acc.at[pl.ds(s * _ZPT, _ZPT)])
    plsc.subcore_barrier()

    pltpu.sync_copy(gidx.at[c, s], gidx_v)
    pltpu.sync_copy(sidx.at[c, s], sidx_v)

    def gather(j, buf, sem):
        pltpu.async_copy(tab.at[gidx_v.at[j]], buf, sem)

    def gather_wait(j, buf, sem):
        pltpu.make_async_copy(tab.at[gidx_v.at[j]], buf, sem).wait()

    def scatter_add(j, buf):
        pltpu.sync_copy(buf, acc.at[sidx_v.at[j]], add=True)

    gather(0, rows_a, sem_a)

    def body(i, _):
        j = 2 * i
        gather_wait(j, rows_a, sem_a)
        gather(j + 1, rows_b, sem_b)
        scatter_add(j, rows_a)
        gather_wait(j + 1, rows_b, sem_b)

        @pl.when(j + 2 < _NCHUNK)
        def _():
            gather(j + 2, rows_a, sem_a)

        scatter_add(j + 1, rows_b)
        return 0

    lax.fori_loop(0, _NCHUNK // 2, body, 0)

    plsc.subcore_barrier()
    pltpu.sync_copy(acc.at[pl.ds(s * _OPT, _OPT)],
                    out.at[c, pl.ds(s * _OPT, _OPT)])


def _sc_aggregate(tab, gidx4, sidx4, zrows):
    mesh = plsc.VectorSubcoreMesh(core_axis_name="c", subcore_axis_name="s")
    return pl.kernel(
        _sc_body,
        out_type=jax.ShapeDtypeStruct((_NC, _NHP, _D), jnp.float32),
        mesh=mesh,
        scratch_types=[
            pltpu.VMEM((_NCHUNK, _CH), jnp.int32),
            pltpu.VMEM((_NCHUNK, _CH), jnp.int32),
            pltpu.VMEM((_CH, _D), jnp.float32),
            pltpu.VMEM((_CH, _D), jnp.float32),
            pltpu.VMEM_SHARED((_ROWS, _D), jnp.float32),
            pltpu.SemaphoreType.DMA,
            pltpu.SemaphoreType.DMA,
        ],
    )(tab, gidx4, sidx4, zrows)


def _tc_body(x_ref, aggw_ref, aggv_ref, wself_ref, b2_ref, wff2_ref,
             bff_ref, qs_ref, nt_ref, asg_ref, gid_ref, out_ref):
    i = pl.program_id(0)
    x = x_ref[...]

    hw = jnp.dot(x, wself_ref[0], preferred_element_type=jnp.float32)
    hv = jnp.dot(x, wself_ref[1], preferred_element_type=jnp.float32)
    hw = jnp.maximum(hw + aggw_ref[0] + b2_ref[0], 0.0)
    hv = jnp.maximum(hv + aggv_ref[0] + b2_ref[1], 0.0)

    w = jnp.abs(jnp.sum(hw * wff2_ref[0], axis=1, keepdims=True)
                + bff_ref[0, 0])
    v = (jnp.sum(hv * wff2_ref[1], axis=1, keepdims=True)
         + bff_ref[0, 1])

    mask = (nt_ref[...] == 1) & (asg_ref[...] == 1)
    s = jnp.where(mask, w * qs_ref[...] + v, 0.0)

    cols = lax.broadcasted_iota(jnp.int32, (_NB, _D), 1)
    onehot = cols == gid_ref[...]
    part = jnp.sum(jnp.where(onehot, s, 0.0), axis=0, keepdims=True)
    part8 = jnp.broadcast_to(part, (8, _D))

    @pl.when(i == 0)
    def _():
        out_ref[...] = part8

    @pl.when(i > 0)
    def _():
        out_ref[...] += part8


def _tc_finish(x, aggw, aggv, wself2, b2, wff2, bff2, qs2, nt2, asg2, gid2):
    full = lambda shape: pl.BlockSpec(shape, lambda i: (0,) * len(shape))
    agg_spec = pl.BlockSpec((1, _NB, _D), lambda i: (i // 5, i % 5, 0))
    col_spec = pl.BlockSpec((_NB, 1), lambda i: (i, 0))
    return pl.pallas_call(
        _tc_body,
        grid=(_NBLK,),
        in_specs=[
            pl.BlockSpec((_NB, _D), lambda i: (i, 0)),
            agg_spec,
            agg_spec,
            full((2, _D, _D)),
            full((2, 1, _D)),
            full((2, 1, _D)),
            pl.BlockSpec(memory_space=pltpu.SMEM),
            col_spec,
            col_spec,
            col_spec,
            col_spec,
        ],
        out_specs=pl.BlockSpec((8, _D), lambda i: (0, 0)),
        out_shape=jax.ShapeDtypeStruct((8, _D), jnp.float32),
    )(x, aggw, aggv, wself2, b2, wff2, bff2, qs2, nt2, asg2, gid2)


def kernel(node_feature, qs, edge_index, edge_type, node_type, assignment,
           graph_ids, Wself_w, Wrel_w, b_w, Wff_w, bff_w,
           Wself_v, Wrel_v, b_v, Wff_v, bff_v):
    src = edge_index[0]
    dst = edge_index[1]
    gidx = (edge_type * _NP + src).astype(jnp.int32)
    half = dst // _NH
    local = (dst - half * _NH).astype(jnp.int32)
    dump = (_NHP + (dst % _NDUMP)).astype(jnp.int32)
    sidx0 = jnp.where(half == 0, local, dump)
    sidx1 = jnp.where(half == 1, local, dump)

    c0 = jnp.cumsum(jnp.where(half == 0, 1, 0))
    c1 = jnp.cumsum(half)
    pos = jnp.where(half == 0, c0 - 1, _E - c1)
    eid = jnp.zeros((_E,), jnp.int32).at[pos].set(
        lax.iota(jnp.int32, _E), mode="drop")
    gidx_p = jnp.take(gidx, eid)
    sidx0_p = jnp.take(sidx0, eid)
    sidx1_p = jnp.take(sidx1, eid)
    gidx4 = jnp.stack([gidx_p[:_ESC], gidx_p[_E - _ESC:]])
    gidx4 = gidx4.reshape(_NC, _NT, _NCHUNK, _CH)
    sidx4 = jnp.stack([sidx0_p[:_ESC], sidx1_p[_E - _ESC:]])
    sidx4 = sidx4.reshape(_NC, _NT, _NCHUNK, _CH)
    zrows = jnp.zeros((_ZPT, _D), jnp.float32)

    wrel2 = jnp.stack([Wrel_w, Wrel_v])
    tw, tv = _tc_prep(node_feature, wrel2)
    tw = tw.reshape(_R * _NP, _D)
    tv = tv.reshape(_R * _NP, _D)

    aggw = _sc_aggregate(tw, gidx4, sidx4, zrows)
    aggv = _sc_aggregate(tv, gidx4, sidx4, zrows)

    wself2 = jnp.stack([Wself_w, Wself_v])
    b2 = jnp.stack([b_w, b_v]).reshape(2, 1, _D)
    wff2 = jnp.stack([Wff_w[:, 0], Wff_v[:, 0]]).reshape(2, 1, _D)
    bff2 = jnp.stack([bff_w[0], bff_v[0]]).reshape(1, 2)

    qs2 = qs.reshape(_N, 1)
    nt2 = node_type.reshape(_N, 1)
    asg2 = assignment.reshape(_N, 1)
    gid2 = graph_ids.reshape(_N, 1)

    out8 = _tc_finish(node_feature, aggw, aggv, wself2, b2, wff2, bff2,
                      qs2, nt2, asg2, gid2)
    return out8[0, :_G]


# speedup vs baseline: 3.4365x; 3.4365x over previous
import jax
import jax.numpy as jnp
from jax import lax
from jax.experimental import pallas as pl
from jax.experimental.pallas import tpu as pltpu
from jax.experimental.pallas import tpu_sc as plsc

_N = 10000
_E = 320000
_D = 128
_R = 3
_G = 100
_NP = 10240
_NC = 2
_NT = 16
_NH = _N // _NC
_NHP = 5120
_CH = 125
_EPT = _E // _NT
_NCHUNK = _EPT // _CH
_NDUMP = 128
_ROWS = _NHP + _NDUMP
_ZPT = _ROWS // _NT
_OPT = _NHP // _NT
_NB = 1000
_NBLK = _N // _NB
_NBP = 1024
_TBLK = _NP // _NBP
_NRING = 4
_NSTAGE = 4
_SCHUNK = _NCHUNK // _NSTAGE


def _prep_body(x_ref, wr_ref, tw_ref, tv_ref):
    x = x_ref[...]
    tw_ref[0] = jnp.dot(x, wr_ref[0, 0], preferred_element_type=jnp.float32)
    tv_ref[0] = jnp.dot(x, wr_ref[1, 0], preferred_element_type=jnp.float32)


def _tc_prep(x, wrel2):
    return pl.pallas_call(
        _prep_body,
        grid=(_R, _TBLK),
        in_specs=[
            pl.BlockSpec((_NBP, _D), lambda r, i: (i, 0)),
            pl.BlockSpec((2, 1, _D, _D), lambda r, i: (0, r, 0, 0)),
        ],
        out_specs=[
            pl.BlockSpec((1, _NBP, _D), lambda r, i: (r, i, 0)),
            pl.BlockSpec((1, _NBP, _D), lambda r, i: (r, i, 0)),
        ],
        out_shape=[
            jax.ShapeDtypeStruct((_R, _NP, _D), jnp.float32),
            jax.ShapeDtypeStruct((_R, _NP, _D), jnp.float32),
        ],
    )(x, wrel2)


def _sc_body(tabw, tabv, gidx, sidx, zrows, out,
             gidx_v, sidx_v, buf0, buf1, buf2, buf3, acc,
             gs0, gs1, gs2, gs3, ss0, ss1, ss2, ss3):
    c = lax.axis_index("c")
    s = lax.axis_index("s")
    bufs = (buf0, buf1, buf2, buf3)
    gsems = (gs0, gs1, gs2, gs3)
    ssems = (ss0, ss1, ss2, ss3)

    for b, tab in ((0, tabw), (1, tabv)):
        pltpu.sync_copy(zrows, acc.at[pl.ds(s * _ZPT, _ZPT)])
        plsc.subcore_barrier()

        def gather(j, k, tab=tab):
            pltpu.async_copy(tab.at[gidx_v.at[j]], bufs[k], gsems[k])

        def gather_wait(j, k, tab=tab):
            pltpu.make_async_copy(tab.at[gidx_v.at[j]], bufs[k],
                                  gsems[k]).wait()

        def scatter(j, k):
            pltpu.async_copy(bufs[k], acc.at[sidx_v.at[j]], ssems[k],
                             add=True)

        def scatter_wait(j, k):
            pltpu.make_async_copy(bufs[k], acc.at[sidx_v.at[j]],
                                  ssems[k]).wait()

        for st in range(_NSTAGE):
            pltpu.sync_copy(gidx.at[s, pl.ds(st * _SCHUNK, _SCHUNK)],
                            gidx_v)
            pltpu.sync_copy(sidx.at[c, s, pl.ds(st * _SCHUNK, _SCHUNK)],
                            sidx_v)

            for k in range(_NRING - 1):
                gather(k, k)

            def loop(i, _):
                for k in range(_NRING):
                    j = _NRING * i + k
                    kn = (k + _NRING - 1) % _NRING
                    gather_wait(j, k)
                    scatter(j, k)

                    @pl.when(j + _NRING - 1 < _SCHUNK)
                    def _():
                        @pl.when(j >= 1)
                        def _():
                            scatter_wait(j - 1, kn)

                        gather(j + _NRING - 1, kn)

                return 0

            lax.fori_loop(0, _SCHUNK // _NRING, loop, 0)

            for t in range(_NRING):
                j = _SCHUNK - _NRING + t
                scatter_wait(j, j % _NRING)

        plsc.subcore_barrier()
        pltpu.sync_copy(acc.at[pl.ds(s * _OPT, _OPT)],
                        out.at[b, c, pl.ds(s * _OPT, _OPT)])
        plsc.subcore_barrier()


def _sc_aggregate(tabw, tabv, gidx3, sidx4, zrows):
    mesh = plsc.VectorSubcoreMesh(core_axis_name="c", subcore_axis_name="s")
    return pl.kernel(
        _sc_body,
        out_type=jax.ShapeDtypeStruct((2, _NC, _NHP, _D), jnp.float32),
        mesh=mesh,
        scratch_types=[
            pltpu.VMEM((_SCHUNK, _CH), jnp.int32),
            pltpu.VMEM((_SCHUNK, _CH), jnp.int32),
            pltpu.VMEM((_CH, _D), jnp.float32),
            pltpu.VMEM((_CH, _D), jnp.float32),
            pltpu.VMEM((_CH, _D), jnp.float32),
            pltpu.VMEM((_CH, _D), jnp.float32),
            pltpu.VMEM_SHARED((_ROWS, _D), jnp.float32),
            pltpu.SemaphoreType.DMA,
            pltpu.SemaphoreType.DMA,
            pltpu.SemaphoreType.DMA,
            pltpu.SemaphoreType.DMA,
            pltpu.SemaphoreType.DMA,
            pltpu.SemaphoreType.DMA,
            pltpu.SemaphoreType.DMA,
            pltpu.SemaphoreType.DMA,
        ],
    )(tabw, tabv, gidx3, sidx4, zrows)


def _tc_body(x_ref, aggw_ref, aggv_ref, wself_ref, b2_ref, wff2_ref,
             bff_ref, qs_ref, nt_ref, asg_ref, gid_ref, out_ref):
    i = pl.program_id(0)
    x = x_ref[...]

    hw = jnp.dot(x, wself_ref[0], preferred_element_type=jnp.float32)
    hv = jnp.dot(x, wself_ref[1], preferred_element_type=jnp.float32)
    hw = jnp.maximum(hw + aggw_ref[0] + b2_ref[0], 0.0)
    hv = jnp.maximum(hv + aggv_ref[0] + b2_ref[1], 0.0)

    w = jnp.abs(jnp.sum(hw * wff2_ref[0], axis=1, keepdims=True)
                + bff_ref[0, 0])
    v = (jnp.sum(hv * wff2_ref[1], axis=1, keepdims=True)
         + bff_ref[0, 1])

    mask = (nt_ref[...] == 1) & (asg_ref[...] == 1)
    s = jnp.where(mask, w * qs_ref[...] + v, 0.0)

    cols = lax.broadcasted_iota(jnp.int32, (_NB, _D), 1)
    onehot = cols == gid_ref[...]
    part = jnp.sum(jnp.where(onehot, s, 0.0), axis=0, keepdims=True)
    part8 = jnp.broadcast_to(part, (8, _D))

    @pl.when(i == 0)
    def _():
        out_ref[...] = part8

    @pl.when(i > 0)
    def _():
        out_ref[...] += part8


def _tc_finish(x, aggw, aggv, wself2, b2, wff2, bff2, qs2, nt2, asg2, gid2):
    full = lambda shape: pl.BlockSpec(shape, lambda i: (0,) * len(shape))
    agg_spec = pl.BlockSpec((1, _NB, _D), lambda i: (i // 5, i % 5, 0))
    col_spec = pl.BlockSpec((_NB, 1), lambda i: (i, 0))
    return pl.pallas_call(
        _tc_body,
        grid=(_NBLK,),
        in_specs=[
            pl.BlockSpec((_NB, _D), lambda i: (i, 0)),
            agg_spec,
            agg_spec,
            full((2, _D, _D)),
            full((2, 1, _D)),
            full((2, 1, _D)),
            pl.BlockSpec(memory_space=pltpu.SMEM),
            col_spec,
            col_spec,
            col_spec,
            col_spec,
        ],
        out_specs=pl.BlockSpec((8, _D), lambda i: (0, 0)),
        out_shape=jax.ShapeDtypeStruct((8, _D), jnp.float32),
    )(x, aggw, aggv, wself2, b2, wff2, bff2, qs2, nt2, asg2, gid2)


def kernel(node_feature, qs, edge_index, edge_type, node_type, assignment,
           graph_ids, Wself_w, Wrel_w, b_w, Wff_w, bff_w,
           Wself_v, Wrel_v, b_v, Wff_v, bff_v):
    src = edge_index[0]
    dst = edge_index[1]
    gidx = (edge_type * _NP + src).astype(jnp.int32)
    gidx3 = gidx.reshape(_NT, _NCHUNK, _CH)
    half = dst // _NH
    local = (dst - half * _NH).astype(jnp.int32)
    dump = (_NHP + (dst % _NDUMP)).astype(jnp.int32)
    sidx4 = jnp.stack([jnp.where(half == 0, local, dump),
                       jnp.where(half == 1, local, dump)])
    sidx4 = sidx4.reshape(_NC, _NT, _NCHUNK, _CH)
    zrows = jnp.zeros((_ZPT, _D), jnp.float32)

    wrel2 = jnp.stack([Wrel_w, Wrel_v])
    tw, tv = _tc_prep(node_feature, wrel2)
    tw = tw.reshape(_R * _NP, _D)
    tv = tv.reshape(_R * _NP, _D)

    agg = _sc_aggregate(tw, tv, gidx3, sidx4, zrows)

    wself2 = jnp.stack([Wself_w, Wself_v])
    b2 = jnp.stack([b_w, b_v]).reshape(2, 1, _D)
    wff2 = jnp.stack([Wff_w[:, 0], Wff_v[:, 0]]).reshape(2, 1, _D)
    bff2 = jnp.stack([bff_w[0], bff_v[0]]).reshape(1, 2)

    qs2 = qs.reshape(_N, 1)
    nt2 = node_type.reshape(_N, 1)
    asg2 = assignment.reshape(_N, 1)
    gid2 = graph_ids.reshape(_N, 1)

    out8 = _tc_finish(node_feature, agg[0], agg[1], wself2, b2, wff2, bff2,
                      qs2, nt2, asg2, gid2)
    return out8[0, :_G]


# speedup vs baseline: 3.5621x; 1.0366x over previous
import jax
import jax.numpy as jnp
from jax import lax
from jax.experimental import pallas as pl
from jax.experimental.pallas import tpu as pltpu
from jax.experimental.pallas import tpu_sc as plsc

_N = 10000
_E = 320000
_D = 128
_R = 3
_G = 100
_NP = 10240
_NC = 2
_NT = 16
_NH = _N // _NC
_NHP = 5120
_CH = 125
_EPT = _E // _NT
_NCHUNK = _EPT // _CH
_NDUMP = 128
_ROWS = _NHP + _NDUMP
_ZPT = _ROWS // _NT
_OPT = _NHP // _NT
_NB = 1000
_NBLK = _N // _NB
_NBP = 1024
_TBLK = _NP // _NBP


def _prep_body(x_ref, wr_ref, tw_ref, tv_ref):
    x = x_ref[...]
    tw_ref[0] = jnp.dot(x, wr_ref[0, 0], preferred_element_type=jnp.float32)
    tv_ref[0] = jnp.dot(x, wr_ref[1, 0], preferred_element_type=jnp.float32)


def _tc_prep(x, wrel2):
    return pl.pallas_call(
        _prep_body,
        grid=(_R, _TBLK),
        in_specs=[
            pl.BlockSpec((_NBP, _D), lambda r, i: (i, 0)),
            pl.BlockSpec((2, 1, _D, _D), lambda r, i: (0, r, 0, 0)),
        ],
        out_specs=[
            pl.BlockSpec((1, _NBP, _D), lambda r, i: (r, i, 0)),
            pl.BlockSpec((1, _NBP, _D), lambda r, i: (r, i, 0)),
        ],
        out_shape=[
            jax.ShapeDtypeStruct((_R, _NP, _D), jnp.float32),
            jax.ShapeDtypeStruct((_R, _NP, _D), jnp.float32),
        ],
    )(x, wrel2)


_NRING = 4
_NSTAGE = 2
_SCHUNK = _NCHUNK // _NSTAGE


def _sc_body(tab, gidx, sidx, zrows, out,
             gidx_v, sidx_v, buf0, buf1, buf2, buf3, acc,
             gs0, gs1, gs2, gs3, ss0, ss1, ss2, ss3):
    c = lax.axis_index("c")
    s = lax.axis_index("s")
    bufs = (buf0, buf1, buf2, buf3)
    gsems = (gs0, gs1, gs2, gs3)
    ssems = (ss0, ss1, ss2, ss3)

    pltpu.sync_copy(zrows, acc.at[pl.ds(s * _ZPT, _ZPT)])
    plsc.subcore_barrier()

    def gather(j, k):
        pltpu.async_copy(tab.at[gidx_v.at[j]], bufs[k], gsems[k])

    def gather_wait(j, k):
        pltpu.make_async_copy(tab.at[gidx_v.at[j]], bufs[k],
                              gsems[k]).wait()

    def scatter(j, k):
        pltpu.async_copy(bufs[k], acc.at[sidx_v.at[j]], ssems[k],
                         add=True)

    def scatter_wait(j, k):
        pltpu.make_async_copy(bufs[k], acc.at[sidx_v.at[j]],
                              ssems[k]).wait()

    for st in range(_NSTAGE):
        pltpu.sync_copy(gidx.at[s, pl.ds(st * _SCHUNK, _SCHUNK)], gidx_v)
        pltpu.sync_copy(sidx.at[c, s, pl.ds(st * _SCHUNK, _SCHUNK)], sidx_v)

        for k in range(_NRING - 1):
            gather(k, k)

        def body(i, _):
            for k in range(_NRING):
                j = _NRING * i + k
                kn = (k + _NRING - 1) % _NRING
                gather_wait(j, k)
                scatter(j, k)

                @pl.when(j + _NRING - 1 < _SCHUNK)
                def _():
                    @pl.when(j >= 1)
                    def _():
                        scatter_wait(j - 1, kn)

                    gather(j + _NRING - 1, kn)

            return 0

        lax.fori_loop(0, _SCHUNK // _NRING, body, 0)

        for t in range(_NRING):
            j = _SCHUNK - _NRING + t
            scatter_wait(j, j % _NRING)

    plsc.subcore_barrier()
    pltpu.sync_copy(acc.at[pl.ds(s * _OPT, _OPT)],
                    out.at[c, pl.ds(s * _OPT, _OPT)])


def _sc_aggregate(tab, gidx3, sidx4, zrows):
    mesh = plsc.VectorSubcoreMesh(core_axis_name="c", subcore_axis_name="s")
    return pl.kernel(
        _sc_body,
        out_type=jax.ShapeDtypeStruct((_NC, _NHP, _D), jnp.float32),
        mesh=mesh,
        scratch_types=[
            pltpu.VMEM((_SCHUNK, _CH), jnp.int32),
            pltpu.VMEM((_SCHUNK, _CH), jnp.int32),
            pltpu.VMEM((_CH, _D), jnp.float32),
            pltpu.VMEM((_CH, _D), jnp.float32),
            pltpu.VMEM((_CH, _D), jnp.float32),
            pltpu.VMEM((_CH, _D), jnp.float32),
            pltpu.VMEM_SHARED((_ROWS, _D), jnp.float32),
            pltpu.SemaphoreType.DMA,
            pltpu.SemaphoreType.DMA,
            pltpu.SemaphoreType.DMA,
            pltpu.SemaphoreType.DMA,
            pltpu.SemaphoreType.DMA,
            pltpu.SemaphoreType.DMA,
            pltpu.SemaphoreType.DMA,
            pltpu.SemaphoreType.DMA,
        ],
    )(tab, gidx3, sidx4, zrows)


def _tc_body(x_ref, aggw_ref, aggv_ref, wself_ref, b2_ref, wff2_ref,
             bff_ref, qs_ref, nt_ref, asg_ref, gid_ref, out_ref):
    i = pl.program_id(0)
    x = x_ref[...]

    hw = jnp.dot(x, wself_ref[0], preferred_element_type=jnp.float32)
    hv = jnp.dot(x, wself_ref[1], preferred_element_type=jnp.float32)
    hw = jnp.maximum(hw + aggw_ref[0] + b2_ref[0], 0.0)
    hv = jnp.maximum(hv + aggv_ref[0] + b2_ref[1], 0.0)

    w = jnp.abs(jnp.sum(hw * wff2_ref[0], axis=1, keepdims=True)
                + bff_ref[0, 0])
    v = (jnp.sum(hv * wff2_ref[1], axis=1, keepdims=True)
         + bff_ref[0, 1])

    mask = (nt_ref[...] == 1) & (asg_ref[...] == 1)
    s = jnp.where(mask, w * qs_ref[...] + v, 0.0)

    cols = lax.broadcasted_iota(jnp.int32, (_NB, _D), 1)
    onehot = cols == gid_ref[...]
    part = jnp.sum(jnp.where(onehot, s, 0.0), axis=0, keepdims=True)
    part8 = jnp.broadcast_to(part, (8, _D))

    @pl.when(i == 0)
    def _():
        out_ref[...] = part8

    @pl.when(i > 0)
    def _():
        out_ref[...] += part8


def _tc_finish(x, aggw, aggv, wself2, b2, wff2, bff2, qs2, nt2, asg2, gid2):
    full = lambda shape: pl.BlockSpec(shape, lambda i: (0,) * len(shape))
    agg_spec = pl.BlockSpec((1, _NB, _D), lambda i: (i // 5, i % 5, 0))
    col_spec = pl.BlockSpec((_NB, 1), lambda i: (i, 0))
    return pl.pallas_call(
        _tc_body,
        grid=(_NBLK,),
        in_specs=[
            pl.BlockSpec((_NB, _D), lambda i: (i, 0)),
            agg_spec,
            agg_spec,
            full((2, _D, _D)),
            full((2, 1, _D)),
            full((2, 1, _D)),
            pl.BlockSpec(memory_space=pltpu.SMEM),
            col_spec,
            col_spec,
            col_spec,
            col_spec,
        ],
        out_specs=pl.BlockSpec((8, _D), lambda i: (0, 0)),
        out_shape=jax.ShapeDtypeStruct((8, _D), jnp.float32),
    )(x, aggw, aggv, wself2, b2, wff2, bff2, qs2, nt2, asg2, gid2)


def kernel(node_feature, qs, edge_index, edge_type, node_type, assignment,
           graph_ids, Wself_w, Wrel_w, b_w, Wff_w, bff_w,
           Wself_v, Wrel_v, b_v, Wff_v, bff_v):
    src = edge_index[0]
    dst = edge_index[1]
    gidx3 = (edge_type * _NP + src).astype(jnp.int32)
    gidx3 = gidx3.reshape(_NT, _NCHUNK, _CH)
    half = dst // _NH
    local = (dst - half * _NH).astype(jnp.int32)
    dump = (_NHP + (dst % _NDUMP)).astype(jnp.int32)
    sidx4 = jnp.stack([jnp.where(half == 0, local, dump),
                       jnp.where(half == 1, local, dump)])
    sidx4 = sidx4.reshape(_NC, _NT, _NCHUNK, _CH)
    zrows = jnp.zeros((_ZPT, _D), jnp.float32)

    wrel2 = jnp.stack([Wrel_w, Wrel_v])
    tw, tv = _tc_prep(node_feature, wrel2)
    tw = tw.reshape(_R * _NP, _D)
    tv = tv.reshape(_R * _NP, _D)

    aggw = _sc_aggregate(tw, gidx3, sidx4, zrows)
    aggv = _sc_aggregate(tv, gidx3, sidx4, zrows)

    wself2 = jnp.stack([Wself_w, Wself_v])
    b2 = jnp.stack([b_w, b_v]).reshape(2, 1, _D)
    wff2 = jnp.stack([Wff_w[:, 0], Wff_v[:, 0]]).reshape(2, 1, _D)
    bff2 = jnp.stack([bff_w[0], bff_v[0]]).reshape(1, 2)

    qs2 = qs.reshape(_N, 1)
    nt2 = node_type.reshape(_N, 1)
    asg2 = assignment.reshape(_N, 1)
    gid2 = graph_ids.reshape(_N, 1)

    out8 = _tc_finish(node_feature, aggw, aggv, wself2, b2, wff2, bff2,
                      qs2, nt2, asg2, gid2)
    return out8[0, :_G]
